# Initial kernel scaffold; baseline (speedup 1.0000x reference)
import jax
import jax.numpy as jnp
from jax.experimental import pallas as pl


def kernel(x, table, pos_table):
    raise NotImplementedError("write your pallas kernel here")


# speedup vs baseline: 5.7399x; 5.7399x over previous
import functools

import jax
import jax.numpy as jnp
from jax import lax
from jax.experimental import pallas as pl
from jax.experimental.pallas import tpu as pltpu
from jax.experimental.pallas import tpu_sc as plsc

VOCAB = 100000
EMBED_DIM = 64
MAX_LENGTH = 200
BATCH = 4096
SEQ = 200

_NC = 2
_NS = 16
_NW = _NC * _NS
_SEQ_PER_W = BATCH // _NW
_SPLIT = 128


def _sc_body(x_hbm, table_hbm, pos_hbm, out_hbm,
             idx_a, idx_b, pos_v, rows_v, sem_a, sem_b):
    wid = lax.axis_index("s") * _NC + lax.axis_index("c")

    pltpu.sync_copy(pos_hbm, pos_v)

    def seq_body(i, carry):
        base = (wid * _SEQ_PER_W + i) * SEQ
        pltpu.sync_copy(x_hbm.at[pl.ds(base, _SPLIT)], idx_a)
        pltpu.sync_copy(x_hbm.at[pl.ds(base + _SPLIT, SEQ - _SPLIT)], idx_b)
        cp_a = pltpu.async_copy(table_hbm.at[idx_a],
                                rows_v.at[pl.ds(0, _SPLIT)], sem_a)
        cp_b = pltpu.async_copy(table_hbm.at[idx_b],
                                rows_v.at[pl.ds(_SPLIT, SEQ - _SPLIT)], sem_b)
        cp_a.wait()
        cp_b.wait()

        def add_body(l, c2):
            for q in range(EMBED_DIM // 16):
                sl = pl.ds(q * 16, 16)
                rows_v[l, sl] = rows_v[l, sl] + pos_v[l, sl]
            return c2

        lax.fori_loop(0, SEQ, add_body, 0)
        pltpu.sync_copy(rows_v, out_hbm.at[pl.ds(base, SEQ)])
        return carry

    lax.fori_loop(0, _SEQ_PER_W, seq_body, 0)


@jax.jit
def kernel(x, table, pos_table):
    x_flat = x.reshape(-1).astype(jnp.int32)
    run = pl.kernel(
        _sc_body,
        out_type=jax.ShapeDtypeStruct((BATCH * SEQ, EMBED_DIM), jnp.float32),
        mesh=plsc.VectorSubcoreMesh(core_axis_name="c", subcore_axis_name="s"),
        compiler_params=pltpu.CompilerParams(use_tc_tiling_on_sc=False),
        scratch_types=[
            pltpu.VMEM((_SPLIT,), jnp.int32),
            pltpu.VMEM((SEQ - _SPLIT,), jnp.int32),
            pltpu.VMEM((MAX_LENGTH, EMBED_DIM), jnp.float32),
            pltpu.VMEM((SEQ, EMBED_DIM), jnp.float32),
            pltpu.SemaphoreType.DMA,
            pltpu.SemaphoreType.DMA,
        ],
    )
    out = run(x_flat, table, pos_table)
    return out.reshape(BATCH, SEQ, EMBED_DIM)


# speedup vs baseline: 6.6358x; 1.1561x over previous
import jax
import jax.numpy as jnp
from jax import lax
from jax.experimental import pallas as pl
from jax.experimental.pallas import tpu as pltpu
from jax.experimental.pallas import tpu_sc as plsc

VOCAB = 100000
EMBED_DIM = 64
MAX_LENGTH = 200
BATCH = 4096
SEQ = 200

_NC = 2
_NS = 16
_NW = _NC * _NS
_SEQ_PER_W = BATCH // _NW
_SPLIT = 128
_REM = SEQ - _SPLIT
_NBUF = 4


def _sc_body(x_hbm, table_hbm, pos_hbm, out_hbm,
             idx_a, idx_b, pos_v, rows_v, gsems, wsems):
    wid = lax.axis_index("s") * _NC + lax.axis_index("c")
    seq0 = wid * _SEQ_PER_W

    pltpu.sync_copy(pos_hbm, pos_v)

    def fetch(j, seq):
        base = seq * SEQ
        pltpu.sync_copy(x_hbm.at[pl.ds(base, _SPLIT)], idx_a.at[j])
        pltpu.sync_copy(x_hbm.at[pl.ds(base + _SPLIT, _REM)], idx_b.at[j])
        pltpu.async_copy(table_hbm.at[idx_a.at[j]],
                         rows_v.at[j, pl.ds(0, _SPLIT)], gsems[j])
        pltpu.async_copy(table_hbm.at[idx_b.at[j]],
                         rows_v.at[j, pl.ds(_SPLIT, _REM)], gsems[j])

    def wait_gather(j):
        pltpu.make_async_copy(table_hbm.at[idx_a.at[j]],
                              rows_v.at[j, pl.ds(0, _SPLIT)], gsems[j]).wait()
        pltpu.make_async_copy(table_hbm.at[idx_b.at[j]],
                              rows_v.at[j, pl.ds(_SPLIT, _REM)], gsems[j]).wait()

    def wait_wb(j):
        pltpu.make_async_copy(rows_v.at[j],
                              out_hbm.at[pl.ds(0, SEQ)], wsems[j]).wait()

    def add_pos(j):
        def body(i, c):
            for u in range(8):
                l = i * 8 + u
                for q in range(EMBED_DIM // 16):
                    sl = pl.ds(q * 16, 16)
                    plsc.addupdate(rows_v.at[j, l, sl], pos_v[l, sl])
            return c
        lax.fori_loop(0, SEQ // 8, body, 0)

    for j in range(_NBUF - 1):
        fetch(j, seq0 + j)

    def outer(k, carry):
        for b in range(_NBUF):
            i = k * _NBUF + b
            jf = (b + _NBUF - 1) % _NBUF

            @pl.when(i >= 1)
            def _():
                wait_wb(jf)

            @pl.when(i + _NBUF - 1 < _SEQ_PER_W)
            def _():
                fetch(jf, seq0 + i + _NBUF - 1)

            wait_gather(b)
            add_pos(b)
            pltpu.async_copy(rows_v.at[b],
                             out_hbm.at[pl.ds((seq0 + i) * SEQ, SEQ)],
                             wsems[b])
        return carry

    lax.fori_loop(0, _SEQ_PER_W // _NBUF, outer, 0)
    wait_wb((_SEQ_PER_W - 1) % _NBUF)


@jax.jit
def kernel(x, table, pos_table):
    x_flat = x.reshape(-1).astype(jnp.int32)

    def body(x_h, t_h, p_h, o_h, ia, ib, pv, rv,
             g0, g1, g2, g3, w0, w1, w2, w3):
        _sc_body(x_h, t_h, p_h, o_h, ia, ib, pv, rv,
                 (g0, g1, g2, g3), (w0, w1, w2, w3))

    run = pl.kernel(
        body,
        out_type=jax.ShapeDtypeStruct((BATCH * SEQ, EMBED_DIM), jnp.float32),
        mesh=plsc.VectorSubcoreMesh(core_axis_name="c", subcore_axis_name="s"),
        compiler_params=pltpu.CompilerParams(use_tc_tiling_on_sc=False),
        scratch_types=[
            pltpu.VMEM((_NBUF, _SPLIT), jnp.int32),
            pltpu.VMEM((_NBUF, _REM), jnp.int32),
            pltpu.VMEM((MAX_LENGTH, EMBED_DIM), jnp.float32),
            pltpu.VMEM((_NBUF, SEQ, EMBED_DIM), jnp.float32),
        ] + [pltpu.SemaphoreType.DMA] * (2 * _NBUF),
    )
    out = run(x_flat, table, pos_table)
    return out.reshape(BATCH, SEQ, EMBED_DIM)


# speedup vs baseline: 8.1403x; 1.2267x over previous
import jax
import jax.numpy as jnp
from jax import lax
from jax.experimental import pallas as pl
from jax.experimental.pallas import tpu as pltpu
from jax.experimental.pallas import tpu_sc as plsc

VOCAB = 100000
EMBED_DIM = 64
MAX_LENGTH = 200
BATCH = 4096
SEQ = 200

_NC = 2
_NS = 16
_NW = _NC * _NS
_BW = BATCH // _NW
_NBUF = 4


def _sc_body(xt_hbm, table_hbm, pos_hbm, out_hbm,
             idx_v, pos_v, rows_v, gsems, wsems):
    wid = lax.axis_index("s") * _NC + lax.axis_index("c")
    b0 = wid * _BW

    pltpu.sync_copy(xt_hbm.at[:, pl.ds(b0, _BW)], idx_v)
    pltpu.sync_copy(pos_hbm, pos_v)

    def fetch(j, s):
        pltpu.async_copy(table_hbm.at[idx_v.at[s]], rows_v.at[j], gsems[j])

    def wait_gather(j, s):
        pltpu.make_async_copy(table_hbm.at[idx_v.at[s]],
                              rows_v.at[j], gsems[j]).wait()

    def wait_wb(j):
        pltpu.make_async_copy(rows_v.at[j],
                              out_hbm.at[pl.ds(b0, _BW), 0], wsems[j]).wait()

    def add_pos(j, s):
        pos_q = [pos_v[s, pl.ds(q * 16, 16)] for q in range(EMBED_DIM // 16)]

        def body(i, c):
            for u in range(8):
                t = i * 8 + u
                for q in range(EMBED_DIM // 16):
                    plsc.addupdate(rows_v.at[j, t, pl.ds(q * 16, 16)],
                                   pos_q[q])
            return c
        lax.fori_loop(0, _BW // 8, body, 0)

    for j in range(_NBUF - 1):
        fetch(j, j)

    def outer(k, carry):
        for b in range(_NBUF):
            s = k * _NBUF + b
            jf = (b + _NBUF - 1) % _NBUF

            @pl.when(s >= 1)
            def _():
                wait_wb(jf)

            @pl.when(s + _NBUF - 1 < SEQ)
            def _():
                fetch(jf, s + _NBUF - 1)

            wait_gather(b, s)
            add_pos(b, s)
            pltpu.async_copy(rows_v.at[b],
                             out_hbm.at[pl.ds(b0, _BW), s], wsems[b])
        return carry

    lax.fori_loop(0, SEQ // _NBUF, outer, 0)
    wait_wb((SEQ - 1) % _NBUF)


@jax.jit
def kernel(x, table, pos_table):
    def body(x_h, t_h, p_h, o_h, iv, pv, rv,
             g0, g1, g2, g3, w0, w1, w2, w3):
        _sc_body(x_h, t_h, p_h, o_h, iv, pv, rv,
                 (g0, g1, g2, g3), (w0, w1, w2, w3))

    run = pl.kernel(
        body,
        out_type=jax.ShapeDtypeStruct((BATCH, SEQ, EMBED_DIM), jnp.float32),
        mesh=plsc.VectorSubcoreMesh(core_axis_name="c", subcore_axis_name="s"),
        compiler_params=pltpu.CompilerParams(use_tc_tiling_on_sc=False),
        scratch_types=[
            pltpu.VMEM((SEQ, _BW), jnp.int32),
            pltpu.VMEM((MAX_LENGTH, EMBED_DIM), jnp.float32),
            pltpu.VMEM((_NBUF, _BW, EMBED_DIM), jnp.float32),
        ] + [pltpu.SemaphoreType.DMA] * (2 * _NBUF),
    )
    xt = jnp.transpose(x.astype(jnp.int32))
    return run(xt, table, pos_table)
